# baseline (speedup 1.0000x reference)
import functools

import jax
import jax.numpy as jnp
from jax import lax
from jax.experimental import pallas as pl
from jax.experimental.pallas import tpu as pltpu
from jax.experimental.pallas import tpu_sc as plsc

N_NODES = 100000
N_EDGES = 6400000
LANES = 128
ROWS = N_EDGES // LANES
RCH = 8
NSUB = 16
NCORE = 2
NW = NSUB * NCORE
N_PAD = 102400
SLICE = N_PAD // NSUB
F32 = jnp.float32

RPW = 1568
LAST_RPW = ROWS - (NW - 1) * RPW
NBUF = 4
assert RPW % RCH == 0 and LAST_RPW % RCH == 0
assert (RPW // RCH) % 4 == 0 and (LAST_RPW // RCH) % 4 == 2

_mesh = plsc.VectorSubcoreMesh(core_axis_name="c", subcore_axis_name="s")


def _worker():
    c = lax.axis_index("c")
    s = lax.axis_index("s")
    w = c * NSUB + s
    nck = jnp.where(w == NW - 1, LAST_RPW // RCH, RPW // RCH)
    return c, s, w, w * RPW, nck


@functools.partial(
    pl.kernel,
    out_type=jax.ShapeDtypeStruct((NCORE, NSUB, N_PAD), F32),
    mesh=_mesh,
    compiler_params=pltpu.CompilerParams(needs_layout_passes=False),
    scratch_types=(
        [pltpu.VMEM((RCH, LANES), jnp.int32)] * NBUF
        + [pltpu.VMEM((N_PAD,), F32)]
        + [pltpu.SemaphoreType.DMA] * NBUF
    ),
)
def _sc_degree(dst_hbm, out_hbm,
               d0, d1, d2, d3, hist_vm,
               li0, li1, li2, li3):
    c, s, w, lo_row, nck = _worker()
    zero16 = jnp.zeros((16,), F32)

    def zbody(i, _):
        hist_vm[pl.ds(i * 16, 16)] = zero16
        return 0

    lax.fori_loop(0, N_PAD // 16, zbody, 0)

    didx = (d0, d1, d2, d3)
    semi = (li0, li1, li2, li3)
    one16 = jnp.ones((16,), F32)

    def load(row, b):
        pltpu.async_copy(dst_hbm.at[pl.ds(row, RCH)], didx[b], semi[b])

    def wait_load(i, b):
        pltpu.make_async_copy(dst_hbm.at[pl.ds(0, RCH)], didx[b],
                              semi[b]).wait()

    def chunk_step(i, b):
        @pl.when(i + 2 < nck)
        def _():
            load(lo_row + (i + 2) * RCH, (b + 2) % NBUF)

        wait_load(i, b)
        for j in range(RCH):
            for v in range(LANES // 16):
                idx = didx[b][j, pl.ds(v * 16, 16)]
                plsc.addupdate_scatter(hist_vm, [idx], one16)

    load(lo_row, 0)
    load(lo_row + RCH, 1)

    def body(k, _):
        for b in range(NBUF):
            chunk_step(NBUF * k + b, b)
        return 0

    lax.fori_loop(0, nck // NBUF, body, 0)

    @pl.when(nck % NBUF == 2)
    def _rem():
        chunk_step(nck - 2, 0)
        chunk_step(nck - 1, 1)

    pltpu.sync_copy(hist_vm, out_hbm.at[c, s, pl.ds(0, N_PAD)])


@functools.partial(
    pl.kernel,
    out_type=jax.ShapeDtypeStruct((NCORE, NSUB, SLICE), F32),
    mesh=_mesh,
    compiler_params=pltpu.CompilerParams(needs_layout_passes=False),
    scratch_types=(
        [pltpu.VMEM((RCH, LANES), jnp.int32)] * (2 * NBUF)
        + [pltpu.VMEM((RCH, LANES), F32)] * 2
        + [pltpu.VMEM((N_PAD,), F32)]
        + [pltpu.VMEM_SHARED((N_PAD,), F32)]
        + [pltpu.SemaphoreType.DMA] * (NBUF + 2)
    ),
)
def _sc_gs(src_hbm, dst_hbm, u_hbm, zeros_hbm, out_hbm,
           s0, s1, s2, s3, d0, d1, d2, d3, va, vb,
           u_vm, acc_sh,
           li0, li1, li2, li3, sc0, sc1):
    c, s, w, lo_row, nck = _worker()
    pltpu.sync_copy(zeros_hbm.at[s], acc_sh.at[pl.ds(s * SLICE, SLICE)])
    pltpu.sync_copy(u_hbm, u_vm)
    plsc.subcore_barrier()

    sidx = (s0, s1, s2, s3)
    didx = (d0, d1, d2, d3)
    vals = (va, vb)
    semi = (li0, li1, li2, li3)
    sems = (sc0, sc1)

    def load(row, b):
        pltpu.async_copy(src_hbm.at[pl.ds(row, RCH)], sidx[b], semi[b])
        pltpu.async_copy(dst_hbm.at[pl.ds(row, RCH)], didx[b], semi[b])

    def wait_load(i, b):
        pltpu.make_async_copy(src_hbm.at[pl.ds(0, RCH)], sidx[b],
                              semi[b]).wait()
        pltpu.make_async_copy(dst_hbm.at[pl.ds(0, RCH)], didx[b],
                              semi[b]).wait()

    def drain_scatter(i, b):
        for j in range(RCH):
            pltpu.make_async_copy(vals[b % 2].at[j],
                                  acc_sh.at[didx[b].at[j]],
                                  sems[b % 2]).wait()

    def gather_issue_scatter(i, b):
        for j in range(RCH):
            for v in range(LANES // 16):
                idx = sidx[b][j, pl.ds(v * 16, 16)]
                vals[b % 2][j, pl.ds(v * 16, 16)] = plsc.load_gather(
                    u_vm, [idx])
        for j in range(RCH):
            pltpu.async_copy(vals[b % 2].at[j], acc_sh.at[didx[b].at[j]],
                             sems[b % 2], add=True)

    def chunk_step(i, b):
        @pl.when(i >= 2)
        def _():
            drain_scatter(i - 2, (b + 2) % NBUF)

        @pl.when(i + 2 < nck)
        def _():
            load(lo_row + (i + 2) * RCH, (b + 2) % NBUF)

        wait_load(i, b)
        gather_issue_scatter(i, b)

    load(lo_row, 0)
    load(lo_row + RCH, 1)

    def body(k, _):
        for b in range(NBUF):
            chunk_step(NBUF * k + b, b)
        return 0

    lax.fori_loop(0, nck // NBUF, body, 0)

    @pl.when(nck % NBUF == 2)
    def _rem():
        chunk_step(nck - 2, 0)
        chunk_step(nck - 1, 1)

    @pl.when(nck % NBUF == 0)
    def _epi0():
        drain_scatter(nck - 2, 2)
        drain_scatter(nck - 1, 3)

    @pl.when(nck % NBUF == 2)
    def _epi2():
        drain_scatter(nck - 2, 0)
        drain_scatter(nck - 1, 1)

    plsc.subcore_barrier()
    pltpu.sync_copy(acc_sh.at[pl.ds(s * SLICE, SLICE)],
                    out_hbm.at[c, s, pl.ds(0, SLICE)])


_R2 = N_PAD // LANES


def _tc1_body(deg_ref, x_ref, u_ref, dinv_ref):
    deg = jnp.full_like(x_ref[...], 1.0)
    for t in range(NW):
        deg = deg + deg_ref[t]
    dinv = lax.rsqrt(deg)
    dinv_ref[...] = dinv
    u_ref[...] = dinv * x_ref[...]


def _tc2_body(g1_ref, u_ref, dinv_ref, w1_ref, b1_ref, w2d_ref, vd_ref):
    dinv = dinv_ref[...]
    u = u_ref[...]
    sarr = dinv * (g1_ref[0] + g1_ref[1] + u)
    t = jnp.zeros_like(sarr)
    for j in range(16):
        h = jnp.maximum(sarr * w1_ref[j] + b1_ref[j], 0.0)
        t = t + h * w2d_ref[j]
    vd_ref[...] = dinv * t


def _tc3_body(gd_ref, vd_ref, dinv_ref, b2d_ref, y0_ref, y1_ref):
    dinv = dinv_ref[...]
    delta = dinv * (gd_ref[0] + gd_ref[1] + vd_ref[...]) + b2d_ref[0]
    sp = jnp.maximum(delta, 0.0) + jnp.log1p(jnp.exp(-jnp.abs(delta)))
    y0_ref[...] = delta - sp
    y1_ref[...] = -sp


def kernel(x, edge_index, W1, b1, W2, b2):
    src = edge_index[0].astype(jnp.int32).reshape(ROWS, LANES)
    dst = edge_index[1].astype(jnp.int32).reshape(ROWS, LANES)
    x_pad = jnp.concatenate(
        [x[:, 0], jnp.zeros((N_PAD - N_NODES,), F32)]).reshape(_R2, LANES)
    zeros1 = jnp.zeros((NSUB, SLICE), F32)

    deg_parts = _sc_degree(dst)
    deg2 = deg_parts.reshape(NW, _R2, LANES)

    u2, dinv2 = pl.pallas_call(
        _tc1_body,
        out_shape=[jax.ShapeDtypeStruct((_R2, LANES), F32)] * 2,
    )(deg2, x_pad)

    g1_parts = _sc_gs(src, dst, u2.reshape(N_PAD), zeros1)
    g12 = g1_parts.reshape(NCORE, _R2, LANES)

    smem_spec = pl.BlockSpec(memory_space=pltpu.SMEM)
    vd2, = pl.pallas_call(
        _tc2_body,
        in_specs=[pl.BlockSpec(), pl.BlockSpec(), pl.BlockSpec(),
                  smem_spec, smem_spec, smem_spec],
        out_shape=[jax.ShapeDtypeStruct((_R2, LANES), F32)],
    )(g12, u2, dinv2, W1.reshape(16), b1, W2[:, 0] - W2[:, 1])

    gd_parts = _sc_gs(src, dst, vd2.reshape(N_PAD), zeros1)
    gd2 = gd_parts.reshape(NCORE, _R2, LANES)

    y0, y1 = pl.pallas_call(
        _tc3_body,
        in_specs=[pl.BlockSpec(), pl.BlockSpec(), pl.BlockSpec(), smem_spec],
        out_shape=[jax.ShapeDtypeStruct((_R2, LANES), F32)] * 2,
    )(gd2, vd2, dinv2, (b2[0] - b2[1]).reshape(1))

    return jnp.stack([y0.reshape(N_PAD)[:N_NODES],
                      y1.reshape(N_PAD)[:N_NODES]], axis=1)


# speedup vs baseline: 1.1913x; 1.1913x over previous
import functools

import jax
import jax.numpy as jnp
from jax import lax
from jax.experimental import pallas as pl
from jax.experimental.pallas import tpu as pltpu
from jax.experimental.pallas import tpu_sc as plsc

N_NODES = 100000
N_EDGES = 6400000
LANES = 128
ROWS = N_EDGES // LANES
RCH = 8
NSUB = 16
NCORE = 2
NW = NSUB * NCORE
N_PAD = 102400
SLICE = N_PAD // NSUB
F32 = jnp.float32

RPW = 1568
LAST_RPW = ROWS - (NW - 1) * RPW
NBUF = 4
assert RPW % RCH == 0 and LAST_RPW % RCH == 0
assert (RPW // RCH) % 4 == 0 and (LAST_RPW // RCH) % 4 == 2

_mesh = plsc.VectorSubcoreMesh(core_axis_name="c", subcore_axis_name="s")


def _worker():
    c = lax.axis_index("c")
    s = lax.axis_index("s")
    w = c * NSUB + s
    nck = jnp.where(w == NW - 1, LAST_RPW // RCH, RPW // RCH)
    return c, s, w, w * RPW, nck


@functools.partial(
    pl.kernel,
    out_type=jax.ShapeDtypeStruct((NCORE, NSUB, SLICE), F32),
    mesh=_mesh,
    scratch_types=(
        [pltpu.VMEM((RCH, LANES), jnp.int32)] * NBUF
        + [pltpu.VMEM((LANES,), F32), pltpu.VMEM_SHARED((N_PAD,), F32)]
        + [pltpu.SemaphoreType.DMA] * (NBUF + 2)
    ),
)
def _sc_degree(dst_hbm, zeros_hbm, out_hbm,
               d0, d1, d2, d3, ones_v, acc_sh,
               li0, li1, li2, li3, sc0, sc1):
    c, s, w, lo_row, nck = _worker()
    one16 = jnp.ones((16,), F32)
    for j in range(LANES // 16):
        ones_v[pl.ds(j * 16, 16)] = one16
    pltpu.sync_copy(zeros_hbm.at[s], acc_sh.at[pl.ds(s * SLICE, SLICE)])
    plsc.subcore_barrier()

    didx = (d0, d1, d2, d3)
    semi = (li0, li1, li2, li3)
    sems = (sc0, sc1)

    def load(row, b):
        pltpu.async_copy(dst_hbm.at[pl.ds(row, RCH)], didx[b], semi[b])

    def wait_load(i, b):
        pltpu.make_async_copy(dst_hbm.at[pl.ds(0, RCH)], didx[b],
                              semi[b]).wait()

    def drain_scatter(i, b):
        for j in range(RCH):
            pltpu.make_async_copy(ones_v, acc_sh.at[didx[b].at[j]],
                                  sems[b % 2]).wait()

    def issue_scatter(i, b):
        for j in range(RCH):
            pltpu.async_copy(ones_v, acc_sh.at[didx[b].at[j]], sems[b % 2],
                             add=True)

    def chunk_step(i, b):
        @pl.when(i >= 2)
        def _():
            drain_scatter(i - 2, (b + 2) % NBUF)

        @pl.when(i + 2 < nck)
        def _():
            load(lo_row + (i + 2) * RCH, (b + 2) % NBUF)

        wait_load(i, b)
        issue_scatter(i, b)

    load(lo_row, 0)
    load(lo_row + RCH, 1)

    def body(k, _):
        for b in range(NBUF):
            chunk_step(NBUF * k + b, b)
        return 0

    lax.fori_loop(0, nck // NBUF, body, 0)

    @pl.when(nck % NBUF == 2)
    def _rem():
        chunk_step(nck - 2, 0)
        chunk_step(nck - 1, 1)

    @pl.when(nck % NBUF == 0)
    def _epi0():
        drain_scatter(nck - 2, 2)
        drain_scatter(nck - 1, 3)

    @pl.when(nck % NBUF == 2)
    def _epi2():
        drain_scatter(nck - 2, 0)
        drain_scatter(nck - 1, 1)

    plsc.subcore_barrier()
    pltpu.sync_copy(acc_sh.at[pl.ds(s * SLICE, SLICE)],
                    out_hbm.at[c, s, pl.ds(0, SLICE)])


@functools.partial(
    pl.kernel,
    out_type=jax.ShapeDtypeStruct((NCORE, NSUB, SLICE), F32),
    mesh=_mesh,
    compiler_params=pltpu.CompilerParams(needs_layout_passes=False),
    scratch_types=(
        [pltpu.VMEM((RCH, LANES), jnp.int32)] * (2 * NBUF)
        + [pltpu.VMEM((RCH, LANES), F32)] * 2
        + [pltpu.VMEM((N_PAD,), F32)]
        + [pltpu.VMEM_SHARED((N_PAD,), F32)]
        + [pltpu.SemaphoreType.DMA] * (NBUF + 2)
    ),
)
def _sc_gs(src_hbm, dst_hbm, u_hbm, zeros_hbm, out_hbm,
           s0, s1, s2, s3, d0, d1, d2, d3, va, vb,
           u_vm, acc_sh,
           li0, li1, li2, li3, sc0, sc1):
    c, s, w, lo_row, nck = _worker()
    pltpu.sync_copy(zeros_hbm.at[s], acc_sh.at[pl.ds(s * SLICE, SLICE)])
    pltpu.sync_copy(u_hbm, u_vm)
    plsc.subcore_barrier()

    sidx = (s0, s1, s2, s3)
    didx = (d0, d1, d2, d3)
    vals = (va, vb)
    semi = (li0, li1, li2, li3)
    sems = (sc0, sc1)

    def load(row, b):
        pltpu.async_copy(src_hbm.at[pl.ds(row, RCH)], sidx[b], semi[b])
        pltpu.async_copy(dst_hbm.at[pl.ds(row, RCH)], didx[b], semi[b])

    def wait_load(i, b):
        pltpu.make_async_copy(src_hbm.at[pl.ds(0, RCH)], sidx[b],
                              semi[b]).wait()
        pltpu.make_async_copy(dst_hbm.at[pl.ds(0, RCH)], didx[b],
                              semi[b]).wait()

    def drain_scatter(i, b):
        for j in range(RCH):
            pltpu.make_async_copy(vals[b % 2].at[j],
                                  acc_sh.at[didx[b].at[j]],
                                  sems[b % 2]).wait()

    def gather_issue_scatter(i, b):
        for j in range(RCH):
            for v in range(LANES // 16):
                idx = sidx[b][j, pl.ds(v * 16, 16)]
                vals[b % 2][j, pl.ds(v * 16, 16)] = plsc.load_gather(
                    u_vm, [idx])
        for j in range(RCH):
            pltpu.async_copy(vals[b % 2].at[j], acc_sh.at[didx[b].at[j]],
                             sems[b % 2], add=True)

    def chunk_step(i, b):
        @pl.when(i >= 2)
        def _():
            drain_scatter(i - 2, (b + 2) % NBUF)

        @pl.when(i + 2 < nck)
        def _():
            load(lo_row + (i + 2) * RCH, (b + 2) % NBUF)

        wait_load(i, b)
        gather_issue_scatter(i, b)

    load(lo_row, 0)
    load(lo_row + RCH, 1)

    def body(k, _):
        for b in range(NBUF):
            chunk_step(NBUF * k + b, b)
        return 0

    lax.fori_loop(0, nck // NBUF, body, 0)

    @pl.when(nck % NBUF == 2)
    def _rem():
        chunk_step(nck - 2, 0)
        chunk_step(nck - 1, 1)

    @pl.when(nck % NBUF == 0)
    def _epi0():
        drain_scatter(nck - 2, 2)
        drain_scatter(nck - 1, 3)

    @pl.when(nck % NBUF == 2)
    def _epi2():
        drain_scatter(nck - 2, 0)
        drain_scatter(nck - 1, 1)

    plsc.subcore_barrier()
    pltpu.sync_copy(acc_sh.at[pl.ds(s * SLICE, SLICE)],
                    out_hbm.at[c, s, pl.ds(0, SLICE)])


_R2 = N_PAD // LANES


def _tc1_body(deg_ref, x_ref, u_ref, dinv_ref):
    deg = deg_ref[0] + deg_ref[1] + 1.0
    dinv = lax.rsqrt(deg)
    dinv_ref[...] = dinv
    u_ref[...] = dinv * x_ref[...]


def _tc2_body(g1_ref, u_ref, dinv_ref, w1_ref, b1_ref, w2d_ref, vd_ref):
    dinv = dinv_ref[...]
    u = u_ref[...]
    sarr = dinv * (g1_ref[0] + g1_ref[1] + u)
    t = jnp.zeros_like(sarr)
    for j in range(16):
        h = jnp.maximum(sarr * w1_ref[j] + b1_ref[j], 0.0)
        t = t + h * w2d_ref[j]
    vd_ref[...] = dinv * t


def _tc3_body(gd_ref, vd_ref, dinv_ref, b2d_ref, y0_ref, y1_ref):
    dinv = dinv_ref[...]
    delta = dinv * (gd_ref[0] + gd_ref[1] + vd_ref[...]) + b2d_ref[0]
    sp = jnp.maximum(delta, 0.0) + jnp.log1p(jnp.exp(-jnp.abs(delta)))
    y0_ref[...] = delta - sp
    y1_ref[...] = -sp


def kernel(x, edge_index, W1, b1, W2, b2):
    src = edge_index[0].astype(jnp.int32).reshape(ROWS, LANES)
    dst = edge_index[1].astype(jnp.int32).reshape(ROWS, LANES)
    x_pad = jnp.concatenate(
        [x[:, 0], jnp.zeros((N_PAD - N_NODES,), F32)]).reshape(_R2, LANES)
    zeros1 = jnp.zeros((NSUB, SLICE), F32)

    deg_parts = _sc_degree(dst, zeros1)
    deg2 = deg_parts.reshape(NCORE, _R2, LANES)

    u2, dinv2 = pl.pallas_call(
        _tc1_body,
        out_shape=[jax.ShapeDtypeStruct((_R2, LANES), F32)] * 2,
    )(deg2, x_pad)

    g1_parts = _sc_gs(src, dst, u2.reshape(N_PAD), zeros1)
    g12 = g1_parts.reshape(NCORE, _R2, LANES)

    smem_spec = pl.BlockSpec(memory_space=pltpu.SMEM)
    vd2, = pl.pallas_call(
        _tc2_body,
        in_specs=[pl.BlockSpec(), pl.BlockSpec(), pl.BlockSpec(),
                  smem_spec, smem_spec, smem_spec],
        out_shape=[jax.ShapeDtypeStruct((_R2, LANES), F32)],
    )(g12, u2, dinv2, W1.reshape(16), b1, W2[:, 0] - W2[:, 1])

    gd_parts = _sc_gs(src, dst, vd2.reshape(N_PAD), zeros1)
    gd2 = gd_parts.reshape(NCORE, _R2, LANES)

    y0, y1 = pl.pallas_call(
        _tc3_body,
        in_specs=[pl.BlockSpec(), pl.BlockSpec(), pl.BlockSpec(), smem_spec],
        out_shape=[jax.ShapeDtypeStruct((_R2, LANES), F32)] * 2,
    )(gd2, vd2, dinv2, (b2[0] - b2[1]).reshape(1))

    return jnp.stack([y0.reshape(N_PAD)[:N_NODES],
                      y1.reshape(N_PAD)[:N_NODES]], axis=1)
